# baseline (speedup 1.0000x reference)
import jax
import jax.numpy as jnp
from jax import lax
from jax.experimental import pallas as pl
from jax.experimental.pallas import tpu as pltpu
from jax.experimental.pallas import tpu_sc as plsc

T = 8192
D = 256
W = 512
R = 128
NW = T // W
NB = 384
HB = W // 2
CB = 2048
NCB = T // CB

NWORK = 32
TPW = (NW * NB) // NWORK

T2 = T // 2
G = 8
NBATCH = (TPW * 2) // G
NSLOT = 3
PFD = 2
NSUB = NW * NB * 2

_DEFAULT = lax.Precision.DEFAULT
_HIGHEST = lax.Precision.HIGHEST


def _plan_body(x_ref, wk_ref, m_ref, selodd_ref, start_ref, cnt_ref,
               srcs_ref, prim2_ref, orow2_ref):
    w = pl.program_id(0)
    xw = x_ref[...]
    wk = wk_ref[...]
    keys = lax.dot_general(xw, wk, (((1,), (0,)), ((), ())),
                           precision=_DEFAULT)
    nrm = jnp.sqrt(jnp.sum(keys * keys, axis=1, keepdims=True)) + 1e-6
    kn = keys / nrm
    sf = lax.dot_general(kn, kn, (((1,), (1,)), ((), ())),
                         precision=_DEFAULT)

    ir = lax.broadcasted_iota(jnp.int32, (W, W), 0)
    ic = lax.broadcasted_iota(jnp.int32, (W, W), 1)
    odd_c = (ic % 2) == 1
    even_c = jnp.logical_not(odd_c)

    masked = jnp.where(odd_c, sf, -3.0)
    bs_col = jnp.max(masked, axis=1, keepdims=True)
    bd_col = jnp.min(jnp.where((sf == bs_col) & odd_c, ic, W),
                     axis=1, keepdims=True)
    bdj_col = bd_col // 2

    ir1 = lax.broadcasted_iota(jnp.int32, (W, 1), 0)
    bs_row = jnp.transpose(bs_col)
    gt = (bs_row > bs_col) & even_c
    eqlt = (bs_row == bs_col) & even_c & (ic < ir)
    rank_col = jnp.sum((gt | eqlt).astype(jnp.float32), axis=1, keepdims=True)

    even_col = (ir1 % 2) == 0
    sel_col = even_col & (rank_col < float(R))
    unm_col = even_col & jnp.logical_not(sel_col)
    unm_f = unm_col.astype(jnp.float32)
    sel_f = sel_col.astype(jnp.float32)

    tri = (ic <= ir).astype(jnp.float32)
    pos_col = lax.dot_general(tri, unm_f, (((1,), (0,)), ((), ())),
                              precision=_HIGHEST) - 1.0

    cols = jnp.concatenate(
        [pos_col, sel_f, unm_f, bdj_col.astype(jnp.float32)], axis=1)
    rows = jnp.transpose(cols)
    pos_row = rows[0:1]
    sel_row = rows[1:2] > 0.5
    unm_row = rows[2:3] > 0.5
    bdj_row = rows[3:4]

    r2 = lax.broadcasted_iota(jnp.int32, (NB, W), 0)
    c2 = lax.broadcasted_iota(jnp.int32, (NB, W), 1)
    r2f = r2.astype(jnp.float32)
    cj = c2 // 2
    top = (r2 < R) & unm_row & (pos_row == r2f)
    bot_own = (r2 >= R) & ((c2 % 2) == 1) & (cj == (r2 - R))
    bot_src = (r2 >= R) & sel_row & (bdj_row == (r2f - float(R)))
    m_ref[0] = (top | bot_own | bot_src).astype(jnp.float32)

    selodd_ref[0] = (sel_col | jnp.logical_not(even_col)).astype(jnp.float32)

    tok_row = (lax.broadcasted_iota(jnp.int32, (1, W), 1)
               + w * W).astype(jnp.float32)
    k128 = lax.broadcasted_iota(jnp.int32, (1, R), 1).astype(jnp.float32)
    oh_unm = (unm_col & (pos_col == k128)).astype(jnp.float32)
    prim_top = lax.dot_general(tok_row, oh_unm, (((1,), (0,)), ((), ())),
                               precision=_HIGHEST)
    jb = lax.broadcasted_iota(jnp.int32, (1, HB), 1)
    prim_bot = (w * W + 2 * jb + 1).astype(jnp.float32)
    prim = jnp.concatenate([prim_top, prim_bot], axis=1)

    jb_row = lax.broadcasted_iota(jnp.int32, (1, HB), 1).astype(jnp.float32)
    hit = (sel_col & (bdj_col == jb_row)).astype(jnp.float32)
    cntj = jnp.sum(hit, axis=0, keepdims=True)
    jr = lax.broadcasted_iota(jnp.int32, (HB, HB), 0)
    jc = lax.broadcasted_iota(jnp.int32, (HB, HB), 1)
    ut = (jr < jc).astype(jnp.float32)
    startj = lax.dot_general(cntj, ut, (((1,), (0,)), ((), ())),
                             precision=_HIGHEST)
    zeros128 = jnp.zeros((1, R), jnp.float32)
    start_full = jnp.concatenate([zeros128, startj], axis=1)
    cnt_full = jnp.concatenate([zeros128, cntj], axis=1)

    cmp = (sel_row & ((bdj_row < bdj_col)
                      | ((bdj_row == bdj_col) & (ic < ir)))).astype(jnp.float32)
    sr_col = jnp.sum(cmp, axis=1, keepdims=True)
    oh_src = (sel_col & (sr_col == k128)).astype(jnp.float32)
    srcs = lax.dot_general(tok_row, oh_src, (((1,), (0,)), ((), ())),
                           precision=_HIGHEST)
    srcs_ref[0] = srcs.astype(jnp.int32)

    ar = lax.broadcasted_iota(jnp.int32, (NB, NB), 0)
    bc = lax.broadcasted_iota(jnp.int32, (NB, NB), 1)
    perm = (bc == jnp.where(ar < TPW, 2 * ar, 2 * ar - (NB - 1))
            ).astype(jnp.float32)

    def _permute(vrow):
        return lax.dot_general(vrow, perm, (((1,), (1,)), ((), ())),
                               precision=_HIGHEST)

    prim_p = _permute(prim)
    start_ref[0] = _permute(start_full).astype(jnp.int32)
    cnt_ref[0] = _permute(cnt_full).astype(jnp.int32)
    a1 = lax.broadcasted_iota(jnp.int32, (1, NB), 1)
    orow = (w * NB
            + jnp.where(a1 < TPW, 2 * a1, 2 * a1 - (NB - 1))
            ).astype(jnp.float32)

    er = lax.broadcasted_iota(jnp.int32, (2 * NB, NB), 0)
    ec = lax.broadcasted_iota(jnp.int32, (2 * NB, NB), 1)
    e_h = er // NB
    e_rem = er % NB
    e_bt2 = e_rem // G
    e_k = e_rem % G
    e_c = (e_bt2 % 2).astype(jnp.float32)
    e_a = e_h * TPW + (e_bt2 // 2) * G + e_k
    p3 = (ec == e_a).astype(jnp.float32)

    def _expand(vrow):
        return lax.dot_general(vrow, p3, (((1,), (1,)), ((), ())),
                               precision=_HIGHEST)

    e_c_row = jnp.transpose(e_c[:, 0:1])
    prim2_ref[0] = (2.0 * _expand(prim_p) + e_c_row).astype(jnp.int32)
    orow2_ref[0] = (2.0 * _expand(orow) + e_c_row).astype(jnp.int32)


def _sizes_body(s_ref, sz_ref):
    cb = pl.program_id(1)
    part = jnp.sum(s_ref[...], axis=1, keepdims=True)

    @pl.when(cb == 0)
    def _():
        sz_ref[0] = part

    @pl.when(cb != 0)
    def _():
        sz_ref[0] += part


def _xmerge_body(x_ref, m_ref, sz_ref, selodd_ref, nx_ref):
    xw = x_ref[...]
    mw = m_ref[0]
    sz = sz_ref[0]
    v = jnp.where(selodd_ref[0] > 0.5, sz, 1.0)
    num = lax.dot_general(mw, xw * v, (((1,), (0,)), ((), ())),
                          precision=_HIGHEST)
    den = lax.dot_general(mw, v, (((1,), (0,)), ((), ())),
                          precision=_HIGHEST)
    nx_ref[...] = num / jnp.clip(den, 1e-6)[...]


def _sc_merge_body(s_hbm, start_hbm, cnt_hbm, srcs_hbm, prim2_hbm, orow2_hbm,
                   out_hbm, start_v, cnt_v, srcs_v, prim2_v, oidx_v, bufs,
                   buf_c0, buf_c1, gsem, wsem, csem0, csem1):
    cid = lax.axis_index("c")
    sid = lax.axis_index("s")
    wid = sid * 2 + cid
    base = wid * TPW
    win = wid // 2
    pltpu.sync_copy(start_hbm.at[pl.ds(base, TPW)], start_v.at[pl.ds(0, TPW)])
    pltpu.sync_copy(cnt_hbm.at[pl.ds(base, TPW)], cnt_v.at[pl.ds(0, TPW)])
    pltpu.sync_copy(srcs_hbm.at[pl.ds(win * R, R)], srcs_v.at[pl.ds(0, R)])
    pltpu.sync_copy(prim2_hbm.at[pl.ds(wid * 2 * TPW, 2 * TPW)], prim2_v)
    pltpu.sync_copy(orow2_hbm.at[pl.ds(wid * NBATCH, NBATCH)], oidx_v)

    def scalar(ref, i):
        return ref[pl.ds(i, 16)][0]

    def issue_gather(bt, slot):
        pltpu.async_copy(s_hbm.at[prim2_v.at[pl.ds(G * bt, G)]],
                         bufs.at[slot], gsem.at[slot])

    for b in range(PFD):
        issue_gather(b, b)

    def group(g, carry):
        for b in range(NSLOT):
            bt = g * NSLOT + b
            bb = bt // 2
            ch = bt % 2
            pltpu.make_async_copy(s_hbm.at[pl.ds(0, G)], bufs.at[b],
                                  gsem.at[b]).wait()

            for pos in range(G):
                t = bb * G + pos
                rb = bufs.at[b].at[pos]
                st = scalar(start_v, t)
                cn = scalar(cnt_v, t)

                @pl.when(cn > 0)
                def _():
                    pltpu.async_copy(
                        s_hbm.at[2 * scalar(srcs_v, st) + ch], buf_c0, csem0)

                def add_from(src_buf):
                    def vadd(i, c3):
                        vals = [src_buf[pl.ds(i * 128 + u * 16, 16)]
                                for u in range(8)]
                        for u in range(8):
                            plsc.addupdate(
                                rb.at[pl.ds(i * 128 + u * 16, 16)], vals[u])
                        return c3
                    lax.fori_loop(0, T2 // 128, vadd, 0)

                def cgroup(j, c2):
                    k0 = 2 * j

                    @pl.when(k0 + 1 < cn)
                    def _():
                        pltpu.async_copy(
                            s_hbm.at[2 * scalar(srcs_v, st + k0 + 1) + ch],
                            buf_c1, csem1)

                    pltpu.make_async_copy(s_hbm.at[0], buf_c0, csem0).wait()
                    add_from(buf_c0)

                    @pl.when(k0 + 2 < cn)
                    def _():
                        pltpu.async_copy(
                            s_hbm.at[2 * scalar(srcs_v, st + k0 + 2) + ch],
                            buf_c0, csem0)

                    @pl.when(k0 + 1 < cn)
                    def _():
                        pltpu.make_async_copy(s_hbm.at[0], buf_c1,
                                              csem1).wait()
                        add_from(buf_c1)

                    return c2

                lax.fori_loop(0, (cn + 1) // 2, cgroup, 0)

            pltpu.async_copy(bufs.at[b], out_hbm.at[oidx_v.at[bt]],
                             wsem.at[b])
            b3 = (b + PFD) % NSLOT
            bt3 = bt + PFD

            @pl.when(bt3 < NBATCH)
            def _():
                @pl.when(bt >= NSLOT - PFD)
                def _():
                    pltpu.make_async_copy(bufs.at[b3],
                                          out_hbm.at[pl.ds(0, G)],
                                          wsem.at[b3]).wait()

                issue_gather(bt3, b3)
        return carry

    lax.fori_loop(0, NBATCH // NSLOT, group, 0)
    for b in range(NSLOT):
        pltpu.make_async_copy(bufs.at[b], out_hbm.at[pl.ds(0, G)],
                              wsem.at[b]).wait()


_SC_MERGE_CACHE = []


def _sc_merge():
    if not _SC_MERGE_CACHE:
        _SC_MERGE_CACHE.append(pl.kernel(
            _sc_merge_body,
            mesh=plsc.VectorSubcoreMesh(core_axis_name="c",
                                        subcore_axis_name="s"),
            out_type=jax.ShapeDtypeStruct((NSUB, T2), jnp.float32),
            scratch_types=[
                pltpu.VMEM((TPW + 16,), jnp.int32),
                pltpu.VMEM((TPW + 16,), jnp.int32),
                pltpu.VMEM((R + 16,), jnp.int32),
                pltpu.VMEM((2 * TPW,), jnp.int32),
                pltpu.VMEM((NBATCH, G), jnp.int32),
                pltpu.VMEM((NSLOT, G, T2), jnp.float32),
                pltpu.VMEM((T2,), jnp.float32),
                pltpu.VMEM((T2,), jnp.float32),
                pltpu.SemaphoreType.DMA((NSLOT,)),
                pltpu.SemaphoreType.DMA((NSLOT,)),
                pltpu.SemaphoreType.DMA,
                pltpu.SemaphoreType.DMA,
            ],
        ))
    return _SC_MERGE_CACHE[0]


@jax.jit
def kernel(x, s, Wk):
    m, selodd, start, cnt, srcs, prim2, orow2 = pl.pallas_call(
        _plan_body,
        grid=(NW,),
        in_specs=[
            pl.BlockSpec((W, D), lambda w: (w, 0)),
            pl.BlockSpec((D, D), lambda w: (0, 0)),
        ],
        out_specs=[
            pl.BlockSpec((1, NB, W), lambda w: (w, 0, 0)),
            pl.BlockSpec((1, W, 1), lambda w: (w, 0, 0)),
            pl.BlockSpec((1, 1, NB), lambda w: (w, 0, 0)),
            pl.BlockSpec((1, 1, NB), lambda w: (w, 0, 0)),
            pl.BlockSpec((1, 1, R), lambda w: (w, 0, 0)),
            pl.BlockSpec((1, 1, 2 * NB), lambda w: (w, 0, 0)),
            pl.BlockSpec((1, 1, 2 * NB), lambda w: (w, 0, 0)),
        ],
        out_shape=[
            jax.ShapeDtypeStruct((NW, NB, W), jnp.float32),
            jax.ShapeDtypeStruct((NW, W, 1), jnp.float32),
            jax.ShapeDtypeStruct((NW, 1, NB), jnp.int32),
            jax.ShapeDtypeStruct((NW, 1, NB), jnp.int32),
            jax.ShapeDtypeStruct((NW, 1, R), jnp.int32),
            jax.ShapeDtypeStruct((NW, 1, 2 * NB), jnp.int32),
            jax.ShapeDtypeStruct((NW, 1, 2 * NB), jnp.int32),
        ],
    )(x, Wk)

    s2 = s.reshape(2 * T, T2)
    ns2 = _sc_merge()(s2, start.reshape(-1), cnt.reshape(-1),
                      srcs.reshape(-1), prim2.reshape(-1),
                      orow2.reshape(-1, G))
    ns = ns2.reshape(NW * NB, T)

    sizes = pl.pallas_call(
        _sizes_body,
        grid=(NW, NCB),
        in_specs=[pl.BlockSpec((W, CB), lambda w, cb: (w, cb))],
        out_specs=pl.BlockSpec((1, W, 1), lambda w, cb: (w, 0, 0)),
        out_shape=jax.ShapeDtypeStruct((NW, W, 1), jnp.float32),
    )(s)

    nx = pl.pallas_call(
        _xmerge_body,
        grid=(NW,),
        in_specs=[
            pl.BlockSpec((W, D), lambda w: (w, 0)),
            pl.BlockSpec((1, NB, W), lambda w: (w, 0, 0)),
            pl.BlockSpec((1, W, 1), lambda w: (w, 0, 0)),
            pl.BlockSpec((1, W, 1), lambda w: (w, 0, 0)),
        ],
        out_specs=pl.BlockSpec((NB, D), lambda w: (w, 0)),
        out_shape=jax.ShapeDtypeStruct((NW * NB, D), jnp.float32),
    )(x, m, sizes, selodd)

    return nx, ns


# speedup vs baseline: 2.3142x; 2.3142x over previous
import jax
import jax.numpy as jnp
from jax import lax
from jax.experimental import pallas as pl
from jax.experimental.pallas import tpu as pltpu
from jax.experimental.pallas import tpu_sc as plsc

T = 8192
D = 256
W = 512
R = 128
NW = T // W
NB = 384
HB = W // 2
CB = 2048
NCB = T // CB

NWORK = 32
TPW = (NW * NB) // NWORK

_DEFAULT = lax.Precision.DEFAULT
_HIGHEST = lax.Precision.HIGHEST


def _plan_body(x_ref, wk_ref, m_ref, selodd_ref, prim_ref, start_ref,
               cnt_ref, srcs_ref):
    w = pl.program_id(0)
    xw = x_ref[...]
    wk = wk_ref[...]
    keys = lax.dot_general(xw, wk, (((1,), (0,)), ((), ())),
                           precision=_DEFAULT)
    nrm = jnp.sqrt(jnp.sum(keys * keys, axis=1, keepdims=True)) + 1e-6
    kn = keys / nrm
    sf = lax.dot_general(kn, kn, (((1,), (1,)), ((), ())),
                         precision=_DEFAULT)

    ir = lax.broadcasted_iota(jnp.int32, (W, W), 0)
    ic = lax.broadcasted_iota(jnp.int32, (W, W), 1)
    odd_c = (ic % 2) == 1
    even_c = jnp.logical_not(odd_c)

    masked = jnp.where(odd_c, sf, -3.0)
    bs_col = jnp.max(masked, axis=1, keepdims=True)
    bd_col = jnp.min(jnp.where((sf == bs_col) & odd_c, ic, W),
                     axis=1, keepdims=True)
    bdj_col = bd_col // 2

    ir1 = lax.broadcasted_iota(jnp.int32, (W, 1), 0)
    bs_row = jnp.transpose(bs_col)
    gt = (bs_row > bs_col) & even_c
    eqlt = (bs_row == bs_col) & even_c & (ic < ir)
    rank_col = jnp.sum((gt | eqlt).astype(jnp.float32), axis=1, keepdims=True)

    even_col = (ir1 % 2) == 0
    sel_col = even_col & (rank_col < float(R))
    unm_col = even_col & jnp.logical_not(sel_col)
    unm_f = unm_col.astype(jnp.float32)
    sel_f = sel_col.astype(jnp.float32)

    tri = (ic <= ir).astype(jnp.float32)
    pos_col = lax.dot_general(tri, unm_f, (((1,), (0,)), ((), ())),
                              precision=_HIGHEST) - 1.0

    cols = jnp.concatenate(
        [pos_col, sel_f, unm_f, bdj_col.astype(jnp.float32)], axis=1)
    rows = jnp.transpose(cols)
    pos_row = rows[0:1]
    sel_row = rows[1:2] > 0.5
    unm_row = rows[2:3] > 0.5
    bdj_row = rows[3:4]

    r2 = lax.broadcasted_iota(jnp.int32, (NB, W), 0)
    c2 = lax.broadcasted_iota(jnp.int32, (NB, W), 1)
    r2f = r2.astype(jnp.float32)
    cj = c2 // 2
    top = (r2 < R) & unm_row & (pos_row == r2f)
    bot_own = (r2 >= R) & ((c2 % 2) == 1) & (cj == (r2 - R))
    bot_src = (r2 >= R) & sel_row & (bdj_row == (r2f - float(R)))
    m_ref[0] = (top | bot_own | bot_src).astype(jnp.float32)

    selodd_ref[0] = (sel_col | jnp.logical_not(even_col)).astype(jnp.float32)

    tok_row = (lax.broadcasted_iota(jnp.int32, (1, W), 1)
               + w * W).astype(jnp.float32)
    k128 = lax.broadcasted_iota(jnp.int32, (1, R), 1).astype(jnp.float32)
    oh_unm = (unm_col & (pos_col == k128)).astype(jnp.float32)
    prim_top = lax.dot_general(tok_row, oh_unm, (((1,), (0,)), ((), ())),
                               precision=_HIGHEST)
    jb = lax.broadcasted_iota(jnp.int32, (1, HB), 1)
    prim_bot = (w * W + 2 * jb + 1).astype(jnp.float32)
    prim = jnp.concatenate([prim_top, prim_bot], axis=1)

    jb_row = lax.broadcasted_iota(jnp.int32, (1, HB), 1).astype(jnp.float32)
    hit = (sel_col & (bdj_col == jb_row)).astype(jnp.float32)
    cntj = jnp.sum(hit, axis=0, keepdims=True)
    jr = lax.broadcasted_iota(jnp.int32, (HB, HB), 0)
    jc = lax.broadcasted_iota(jnp.int32, (HB, HB), 1)
    ut = (jr < jc).astype(jnp.float32)
    startj = lax.dot_general(cntj, ut, (((1,), (0,)), ((), ())),
                             precision=_HIGHEST)
    zeros128 = jnp.zeros((1, R), jnp.float32)
    start_full = jnp.concatenate([zeros128, startj], axis=1)
    cnt_full = jnp.concatenate([zeros128, cntj], axis=1)

    cmp = (sel_row & ((bdj_row < bdj_col)
                      | ((bdj_row == bdj_col) & (ic < ir)))).astype(jnp.float32)
    sr_col = jnp.sum(cmp, axis=1, keepdims=True)
    oh_src = (sel_col & (sr_col == k128)).astype(jnp.float32)
    srcs = lax.dot_general(tok_row, oh_src, (((1,), (0,)), ((), ())),
                           precision=_HIGHEST)
    srcs_ref[0] = srcs.astype(jnp.int32)

    ar = lax.broadcasted_iota(jnp.int32, (NB, NB), 0)
    bc = lax.broadcasted_iota(jnp.int32, (NB, NB), 1)
    perm = (bc == jnp.where(ar < TPW, 2 * ar, 2 * ar - (NB - 1))
            ).astype(jnp.float32)

    def _permute(vrow):
        return lax.dot_general(vrow, perm, (((1,), (1,)), ((), ())),
                               precision=_HIGHEST)

    prim_ref[0] = _permute(prim).astype(jnp.int32)
    start_ref[0] = _permute(start_full).astype(jnp.int32)
    cnt_ref[0] = _permute(cnt_full).astype(jnp.int32)


def _sizes_body(s_ref, sz_ref):
    cb = pl.program_id(1)
    part = jnp.sum(s_ref[...], axis=1, keepdims=True)

    @pl.when(cb == 0)
    def _():
        sz_ref[0] = part

    @pl.when(cb != 0)
    def _():
        sz_ref[0] += part


def _xmerge_body(x_ref, m_ref, sz_ref, selodd_ref, nx_ref):
    xw = x_ref[...]
    mw = m_ref[0]
    sz = sz_ref[0]
    v = jnp.where(selodd_ref[0] > 0.5, sz, 1.0)
    num = lax.dot_general(mw, xw * v, (((1,), (0,)), ((), ())),
                          precision=_HIGHEST)
    den = lax.dot_general(mw, v, (((1,), (0,)), ((), ())),
                          precision=_HIGHEST)
    nx_ref[...] = num / jnp.clip(den, 1e-6)[...]


NSLOT = 8
PFD = 4


def _sc_merge_body(s_hbm, prim_hbm, start_hbm, cnt_hbm, srcs_hbm, out_hbm,
                   prim_v, start_v, cnt_v, srcs_v, bufs, buf_c0, buf_c1,
                   gsem, wsem, csem0, csem1):
    cid = lax.axis_index("c")
    sid = lax.axis_index("s")
    wid = sid * 2 + cid
    base = wid * TPW
    win = wid // 2
    half = wid % 2
    pltpu.sync_copy(prim_hbm.at[pl.ds(base, TPW)], prim_v.at[pl.ds(0, TPW)])
    pltpu.sync_copy(start_hbm.at[pl.ds(base, TPW)], start_v.at[pl.ds(0, TPW)])
    pltpu.sync_copy(cnt_hbm.at[pl.ds(base, TPW)], cnt_v.at[pl.ds(0, TPW)])
    pltpu.sync_copy(srcs_hbm.at[pl.ds(win * R, R)], srcs_v.at[pl.ds(0, R)])

    def scalar(ref, i):
        return ref[pl.ds(i, 16)][0]

    for b in range(PFD):
        pltpu.async_copy(s_hbm.at[scalar(prim_v, b)], bufs.at[b], gsem.at[b])

    def group(g, carry):
        for b in range(NSLOT):
            t = g * NSLOT + b
            rb = bufs.at[b]
            st = scalar(start_v, t)
            cn = scalar(cnt_v, t)

            @pl.when(cn > 0)
            def _():
                pltpu.async_copy(s_hbm.at[scalar(srcs_v, st)], buf_c0, csem0)

            pltpu.make_async_copy(s_hbm.at[0], rb, gsem.at[b]).wait()

            def add_from(src_buf):
                def vadd(i, c3):
                    vals = [src_buf[pl.ds(i * 128 + u * 16, 16)]
                            for u in range(8)]
                    for u in range(8):
                        plsc.addupdate(rb.at[pl.ds(i * 128 + u * 16, 16)],
                                       vals[u])
                    return c3
                lax.fori_loop(0, T // 128, vadd, 0)

            def cgroup(j, c2):
                k0 = 2 * j

                @pl.when(k0 + 1 < cn)
                def _():
                    pltpu.async_copy(s_hbm.at[scalar(srcs_v, st + k0 + 1)],
                                     buf_c1, csem1)

                pltpu.make_async_copy(s_hbm.at[0], buf_c0, csem0).wait()
                add_from(buf_c0)

                @pl.when(k0 + 2 < cn)
                def _():
                    pltpu.async_copy(s_hbm.at[scalar(srcs_v, st + k0 + 2)],
                                     buf_c0, csem0)

                @pl.when(k0 + 1 < cn)
                def _():
                    pltpu.make_async_copy(s_hbm.at[0], buf_c1, csem1).wait()
                    add_from(buf_c1)

                return c2

            lax.fori_loop(0, (cn + 1) // 2, cgroup, 0)
            orow = win * NB + 2 * t + half
            pltpu.async_copy(rb, out_hbm.at[orow], wsem.at[b])
            b3 = (b + PFD) % NSLOT
            t3 = t + PFD

            @pl.when(t3 < TPW)
            def _():
                @pl.when(t >= NSLOT - PFD)
                def _():
                    pltpu.make_async_copy(bufs.at[b3], out_hbm.at[0],
                                          wsem.at[b3]).wait()

                pltpu.async_copy(s_hbm.at[scalar(prim_v, t3)], bufs.at[b3],
                                 gsem.at[b3])
        return carry

    lax.fori_loop(0, TPW // NSLOT, group, 0)
    for b in range(NSLOT):
        pltpu.make_async_copy(bufs.at[b], out_hbm.at[0], wsem.at[b]).wait()


_SC_MERGE_CACHE = []


def _sc_merge():
    if not _SC_MERGE_CACHE:
        _SC_MERGE_CACHE.append(pl.kernel(
            _sc_merge_body,
            mesh=plsc.VectorSubcoreMesh(core_axis_name="c",
                                        subcore_axis_name="s"),
            out_type=jax.ShapeDtypeStruct((NW * NB, T), jnp.float32),
            scratch_types=[
                pltpu.VMEM((TPW + 16,), jnp.int32),
                pltpu.VMEM((TPW + 16,), jnp.int32),
                pltpu.VMEM((TPW + 16,), jnp.int32),
                pltpu.VMEM((R + 16,), jnp.int32),
                pltpu.VMEM((NSLOT, T), jnp.float32),
                pltpu.VMEM((T,), jnp.float32),
                pltpu.VMEM((T,), jnp.float32),
                pltpu.SemaphoreType.DMA((NSLOT,)),
                pltpu.SemaphoreType.DMA((NSLOT,)),
                pltpu.SemaphoreType.DMA,
                pltpu.SemaphoreType.DMA,
            ],
        ))
    return _SC_MERGE_CACHE[0]


@jax.jit
def kernel(x, s, Wk):
    m, selodd, prim, start, cnt, srcs = pl.pallas_call(
        _plan_body,
        grid=(NW,),
        in_specs=[
            pl.BlockSpec((W, D), lambda w: (w, 0)),
            pl.BlockSpec((D, D), lambda w: (0, 0)),
        ],
        out_specs=[
            pl.BlockSpec((1, NB, W), lambda w: (w, 0, 0)),
            pl.BlockSpec((1, W, 1), lambda w: (w, 0, 0)),
            pl.BlockSpec((1, 1, NB), lambda w: (w, 0, 0)),
            pl.BlockSpec((1, 1, NB), lambda w: (w, 0, 0)),
            pl.BlockSpec((1, 1, NB), lambda w: (w, 0, 0)),
            pl.BlockSpec((1, 1, R), lambda w: (w, 0, 0)),
        ],
        out_shape=[
            jax.ShapeDtypeStruct((NW, NB, W), jnp.float32),
            jax.ShapeDtypeStruct((NW, W, 1), jnp.float32),
            jax.ShapeDtypeStruct((NW, 1, NB), jnp.int32),
            jax.ShapeDtypeStruct((NW, 1, NB), jnp.int32),
            jax.ShapeDtypeStruct((NW, 1, NB), jnp.int32),
            jax.ShapeDtypeStruct((NW, 1, R), jnp.int32),
        ],
    )(x, Wk)

    ns = _sc_merge()(s, prim.reshape(-1), start.reshape(-1),
                     cnt.reshape(-1), srcs.reshape(-1))

    sizes = pl.pallas_call(
        _sizes_body,
        grid=(NW, NCB),
        in_specs=[pl.BlockSpec((W, CB), lambda w, cb: (w, cb))],
        out_specs=pl.BlockSpec((1, W, 1), lambda w, cb: (w, 0, 0)),
        out_shape=jax.ShapeDtypeStruct((NW, W, 1), jnp.float32),
    )(s)

    nx = pl.pallas_call(
        _xmerge_body,
        grid=(NW,),
        in_specs=[
            pl.BlockSpec((W, D), lambda w: (w, 0)),
            pl.BlockSpec((1, NB, W), lambda w: (w, 0, 0)),
            pl.BlockSpec((1, W, 1), lambda w: (w, 0, 0)),
            pl.BlockSpec((1, W, 1), lambda w: (w, 0, 0)),
        ],
        out_specs=pl.BlockSpec((NB, D), lambda w: (w, 0)),
        out_shape=jax.ShapeDtypeStruct((NW * NB, D), jnp.float32),
    )(x, m, sizes, selodd)

    return nx, ns
